# baseline (speedup 1.0000x reference)
import jax
import jax.numpy as jnp
from jax.experimental import pallas as pl

_B = 512


def _prods(x):
    p0 = x[:, 0, :] * x[:, 1, :] * x[:, 2, :] * x[:, 3, :] * x[:, 4, :]
    p1 = x[:, 5, :] * x[:, 6, :] * x[:, 7, :] * x[:, 8, :] * x[:, 9, :]
    return jnp.stack([p0, p1], axis=1)


def _body(a_ref, b_ref, c_ref, d_ref, o_ref):
    o_ref[0] = _prods(a_ref[0])
    o_ref[1] = _prods(b_ref[0])
    o_ref[2] = _prods(c_ref[0])
    o_ref[3] = _prods(d_ref[0])


def kernel(inputs):
    n, r, d = inputs.shape
    h = n // 4
    x = inputs.reshape(4, h, r, d)
    out = pl.pallas_call(
        _body,
        grid=(h // _B,),
        in_specs=[
            pl.BlockSpec((1, _B, r, d), lambda i: (0, i, 0, 0)),
            pl.BlockSpec((1, _B, r, d), lambda i: (1, i, 0, 0)),
            pl.BlockSpec((1, _B, r, d), lambda i: (2, i, 0, 0)),
            pl.BlockSpec((1, _B, r, d), lambda i: (3, i, 0, 0)),
        ],
        out_specs=pl.BlockSpec((4, _B, 2, d), lambda i: (0, i, 0, 0)),
        out_shape=jax.ShapeDtypeStruct((4, h, 2, d), inputs.dtype),
    )(x, x, x, x)
    return out.reshape(n, 2, d)
